# baseline (speedup 1.0000x reference)
import jax
import jax.numpy as jnp
from jax.experimental import pallas as pl
from jax.experimental.pallas import tpu as pltpu

_NCHUNK = 25


def _dma_copy(x_ref, o_ref, sems):
    R = x_ref.shape[0]
    step = R // _NCHUNK
    if _NCHUNK == 1:
        copies = [pltpu.make_async_copy(x_ref, o_ref, sems.at[0])]
    else:
        copies = [
            pltpu.make_async_copy(
                x_ref.at[pl.ds(i * step, step), :],
                o_ref.at[pl.ds(i * step, step), :],
                sems.at[i],
            )
            for i in range(_NCHUNK)
        ]
    for c in copies:
        c.start()
    for c in copies:
        c.wait()


def kernel(edge_index):
    E = edge_index.shape[1]
    total = 2 * E
    C = 512
    R = total // C
    x = edge_index.reshape(R, C)
    out = pl.pallas_call(
        _dma_copy,
        in_specs=[pl.BlockSpec(memory_space=pl.ANY)],
        out_specs=pl.BlockSpec(memory_space=pl.ANY),
        out_shape=jax.ShapeDtypeStruct((R, C), edge_index.dtype),
        scratch_shapes=[pltpu.SemaphoreType.DMA((_NCHUNK,))],
    )(x)
    return out.reshape(2, E)


# speedup vs baseline: 7.5792x; 7.5792x over previous
import jax
import jax.numpy as jnp
from jax.experimental import pallas as pl
from jax.experimental.pallas import tpu as pltpu

_C = 512
_BR = 5000


def _copy_block(x_ref, o_ref):
    o_ref[...] = x_ref[...]


def kernel(edge_index):
    E = edge_index.shape[1]
    total = 2 * E
    R = total // _C
    x = edge_index.reshape(R, _C)
    out = pl.pallas_call(
        _copy_block,
        grid=(R // _BR,),
        in_specs=[pl.BlockSpec((_BR, _C), lambda i: (i, 0))],
        out_specs=pl.BlockSpec((_BR, _C), lambda i: (i, 0)),
        out_shape=jax.ShapeDtypeStruct((R, _C), edge_index.dtype),
        compiler_params=pltpu.CompilerParams(
            dimension_semantics=("arbitrary",),
        ),
    )(x)
    return out.reshape(2, E)
